# baseline (speedup 1.0000x reference)
import functools

import jax
import jax.numpy as jnp
from jax import lax
from jax.experimental import pallas as pl
from jax.experimental.pallas import tpu as pltpu
from jax.experimental.pallas import tpu_sc as plsc

_EPS = 1e-12
_L = 16


def _lanesum(x):
    lane = lax.broadcasted_iota(jnp.int32, (_L,), 0)
    dnums = lax.GatherDimensionNumbers(
        offset_dims=(), collapsed_slice_dims=(0,), start_index_map=(0,))
    for sh in (8, 4, 2, 1):
        perm = lax.gather(x, (lane ^ sh)[:, None], dnums, slice_sizes=(1,),
                          mode=lax.GatherScatterMode.PROMISE_IN_BOUNDS)
        x = x + perm
    return x


def _rsqrt16(x):
    i = lax.bitcast_convert_type(x, jnp.int32)
    i = jnp.int32(0x5F3759DF) - lax.shift_right_logical(i, 1)
    y = lax.bitcast_convert_type(i, jnp.float32)
    for _ in range(4):
        y = y * (1.5 - 0.5 * x * y * y)
    return y


def _build(B, S, H, V, P, CH):
    NV = H // _L
    NCH = S // CH
    mesh = plsc.VectorSubcoreMesh(core_axis_name="c", subcore_axis_name="s")

    @functools.partial(
        pl.kernel,
        mesh=mesh,
        out_type=jax.ShapeDtypeStruct((B, S, H), jnp.float32),
        scratch_types=[
            pltpu.VMEM((CH,), jnp.int32),
            pltpu.VMEM((CH, H), jnp.float32),
            pltpu.VMEM((CH, H), jnp.float32),
            pltpu.VMEM((H,), jnp.float32),
            pltpu.VMEM((H,), jnp.float32),
            pltpu.VMEM((H,), jnp.float32),
            pltpu.VMEM((P, H), jnp.float32),
            pltpu.SemaphoreType.DMA,
        ],
    )
    def emb_kernel(ids_hbm, word_hbm, pos_hbm, type_hbm, prompt_hbm,
                   gamma_hbm, beta_hbm, out_hbm,
                   idx_v, rows_v, pos_v, type_v, gamma_v, beta_v, prompt_v, sem):
        b = lax.axis_index("s") * 2 + lax.axis_index("c")
        pltpu.sync_copy(type_hbm.at[0], type_v)
        pltpu.sync_copy(gamma_hbm, gamma_v)
        pltpu.sync_copy(beta_hbm, beta_v)
        pltpu.sync_copy(prompt_hbm, prompt_v)

        def chunk_body(c, carry):
            s0 = c * CH
            pltpu.sync_copy(ids_hbm.at[b, pl.ds(s0, CH)], idx_v)
            gather = pltpu.async_copy(word_hbm.at[idx_v], rows_v, sem)
            pltpu.sync_copy(pos_hbm.at[pl.ds(s0, CH)], pos_v)
            gather.wait()

            @pl.when(c == 0)
            def _():
                def pr_body(i, _):
                    r = i // NV
                    o = (i % NV) * _L
                    rows_v[r + 1, pl.ds(o, _L)] = prompt_v[r, pl.ds(o, _L)]
                    return 0
                lax.fori_loop(0, P * NV, pr_body, 0)

            def row_body(r, rcarry):
                def p1(v, acc):
                    o = v * _L
                    x = (rows_v[r, pl.ds(o, _L)] + pos_v[r, pl.ds(o, _L)]
                         + type_v[pl.ds(o, _L)])
                    rows_v[r, pl.ds(o, _L)] = x
                    return acc[0] + x, acc[1] + x * x
                z = jnp.zeros((_L,), jnp.float32)
                acc, acc2 = lax.fori_loop(0, NV, p1, (z, z))
                mv = _lanesum(acc) * (1.0 / H)
                var = jnp.maximum(_lanesum(acc2) * (1.0 / H) - mv * mv, 0.0) + _EPS
                a = _rsqrt16(var)
                ma = mv * a

                def p2(v, _):
                    o = v * _L
                    x = rows_v[r, pl.ds(o, _L)]
                    xh = x * a - ma
                    rows_v[r, pl.ds(o, _L)] = (
                        xh * gamma_v[pl.ds(o, _L)] + beta_v[pl.ds(o, _L)])
                    return 0
                lax.fori_loop(0, NV, p2, 0)
                return rcarry
            lax.fori_loop(0, CH, row_body, 0)
            pltpu.sync_copy(rows_v, out_hbm.at[b, pl.ds(s0, CH)])
            return carry

        lax.fori_loop(0, NCH, chunk_body, 0)

    return emb_kernel


@jax.jit
def kernel(input_ids, word_emb, pos_emb, type_emb, prompt_emb, gamma, beta):
    B, S = input_ids.shape
    V, H = word_emb.shape
    P = prompt_emb.shape[0]
    emb = _build(B, S, H, V, P, CH=64)
    return emb(input_ids, word_emb, pos_emb, type_emb, prompt_emb, gamma, beta)


# speedup vs baseline: 2.0129x; 2.0129x over previous
import functools

import jax
import jax.numpy as jnp
from jax import lax
from jax.experimental import pallas as pl
from jax.experimental.pallas import tpu as pltpu
from jax.experimental.pallas import tpu_sc as plsc

_EPS = 1e-12
_L = 16


def _rsqrt16(x):
    i = lax.bitcast_convert_type(x, jnp.int32)
    i = jnp.full((_L,), 0x5F3759DF, jnp.int32) - lax.shift_right_logical(i, 1)
    y = lax.bitcast_convert_type(i, jnp.float32)
    for _ in range(4):
        y = y * (1.5 - 0.5 * x * y * y)
    return y


def _treesum(vs):
    while len(vs) > 1:
        vs = [a + b for a, b in zip(vs[::2], vs[1::2])]
    return vs[0]


def _build(B, S, H, V, P, CH):
    NV = H // _L
    NCH = S // CH
    mesh = plsc.VectorSubcoreMesh(core_axis_name="c", subcore_axis_name="s")

    @functools.partial(
        pl.kernel,
        mesh=mesh,
        out_type=jax.ShapeDtypeStruct((B, S, H), jnp.float32),
        scratch_types=[
            pltpu.VMEM((CH,), jnp.int32),
            pltpu.VMEM((CH, H), jnp.float32),
            pltpu.VMEM((CH, H), jnp.float32),
            pltpu.VMEM((H,), jnp.float32),
            pltpu.VMEM((H,), jnp.float32),
            pltpu.VMEM((H,), jnp.float32),
            pltpu.VMEM((P, H), jnp.float32),
            pltpu.SMEM((CH,), jnp.float32),
            pltpu.SMEM((CH,), jnp.float32),
            pltpu.SemaphoreType.DMA,
        ],
    )
    def emb_kernel(ids_hbm, word_hbm, pos_hbm, type_hbm, prompt_hbm,
                   gamma_hbm, beta_hbm, out_hbm,
                   idx_v, rows_v, pos_v, type_v, gamma_v, beta_v, prompt_v,
                   a_sm, nma_sm, sem):
        b = lax.axis_index("s") * 2 + lax.axis_index("c")
        pltpu.sync_copy(type_hbm.at[0], type_v)
        pltpu.sync_copy(gamma_hbm, gamma_v)
        pltpu.sync_copy(beta_hbm, beta_v)
        pltpu.sync_copy(prompt_hbm, prompt_v)

        def chunk_body(c, carry):
            s0 = c * CH
            pltpu.sync_copy(ids_hbm.at[b, pl.ds(s0, CH)], idx_v)
            gather = pltpu.async_copy(word_hbm.at[idx_v], rows_v, sem)
            pltpu.sync_copy(pos_hbm.at[pl.ds(s0, CH)], pos_v)
            gather.wait()

            @pl.when(c == 0)
            def _():
                @plsc.parallel_loop(0, P * NV, unroll=4)
                def pr_body(i):
                    r = i // NV
                    o = (i % NV) * _L
                    rows_v[r + 1, pl.ds(o, _L)] = prompt_v[r, pl.ds(o, _L)]

            @plsc.parallel_loop(0, CH)
            def row_body(r):
                z = jnp.zeros((_L,), jnp.float32)
                accs = [z, z, z, z]
                acc2s = [z, z, z, z]
                for v in range(NV):
                    o = v * _L
                    x = (rows_v[r, pl.ds(o, _L)] + pos_v[r, pl.ds(o, _L)]
                         + type_v[pl.ds(o, _L)])
                    rows_v[r, pl.ds(o, _L)] = x
                    accs[v % 4] = accs[v % 4] + x
                    acc2s[v % 4] = acc2s[v % 4] + x * x
                acc = (accs[0] + accs[1]) + (accs[2] + accs[3])
                acc2 = (acc2s[0] + acc2s[1]) + (acc2s[2] + acc2s[3])
                s1 = _treesum([acc[j] for j in range(_L)])
                s2 = _treesum([acc2[j] for j in range(_L)])
                m = s1 * (1.0 / H)
                t = s2 * (1.0 / H) - m * m
                var = jnp.where(t > 0.0, t, 0.0) + _EPS
                i = lax.bitcast_convert_type(var, jnp.int32)
                i = jnp.int32(0x5F3759DF) - lax.shift_right_logical(i, 1)
                y = lax.bitcast_convert_type(i, jnp.float32)
                for _ in range(4):
                    y = y * (1.5 - 0.5 * var * y * y)
                a_sm[r] = y
                nma_sm[r] = -(m * y)

            def col_body(v, carry2):
                o = v * _L
                g = gamma_v[pl.ds(o, _L)]
                be = beta_v[pl.ds(o, _L)]

                @plsc.parallel_loop(0, CH, unroll=4)
                def p2_body(r):
                    x = rows_v[r, pl.ds(o, _L)]
                    xh = x * jnp.full((_L,), a_sm[r]) + jnp.full((_L,), nma_sm[r])
                    rows_v[r, pl.ds(o, _L)] = xh * g + be
                return carry2
            lax.fori_loop(0, NV, col_body, 0)

            pltpu.sync_copy(rows_v, out_hbm.at[b, pl.ds(s0, CH)])
            return carry

        lax.fori_loop(0, NCH, chunk_body, 0)

    return emb_kernel


@jax.jit
def kernel(input_ids, word_emb, pos_emb, type_emb, prompt_emb, gamma, beta):
    B, S = input_ids.shape
    V, H = word_emb.shape
    P = prompt_emb.shape[0]
    emb = _build(B, S, H, V, P, CH=64)
    return emb(input_ids, word_emb, pos_emb, type_emb, prompt_emb, gamma, beta)
